# Initial kernel scaffold; baseline (speedup 1.0000x reference)
import jax
import jax.numpy as jnp
from jax.experimental import pallas as pl


def kernel(x, edge_index, W1, b1, W2, b2):
    raise NotImplementedError("write your pallas kernel here")


# speedup vs baseline: 2.0134x; 2.0134x over previous
import functools

import jax
import jax.numpy as jnp
from jax import lax
from jax.experimental import pallas as pl
from jax.experimental.pallas import tpu as pltpu
from jax.experimental.pallas import tpu_sc as plsc

N = 10000
E = 160000
NFEAT = 256
NHID = 1024
NCLASS = 256

NTILES = 16
NCORES = 2
CHUNK = 128
EPT_CH = 80
EPAD = NTILES * EPT_CH * CHUNK
NPAD = 10240
ROWS_PER_TILE = NPAD // NTILES
CW = 128


def _make_segsum(nchunks):
  cp = nchunks // NCORES
  mesh = plsc.VectorSubcoreMesh(core_axis_name="c", subcore_axis_name="s")

  @functools.partial(
      pl.kernel,
      mesh=mesh,
      out_type=jax.ShapeDtypeStruct((nchunks, NPAD, CW), jnp.float32),
      scratch_types=[
          pltpu.VMEM((EPT_CH, CHUNK), jnp.int32),
          pltpu.VMEM((EPT_CH, CHUNK), jnp.int32),
          pltpu.VMEM((EPT_CH, CHUNK), jnp.int32),
          pltpu.VMEM((CHUNK, CW), jnp.float32),
          pltpu.VMEM_SHARED((NPAD, CW), jnp.float32),
      ],
  )
  def segsum(table, src3, dst3, zrows, out_r, src_v, dst_v, gidx_v, gbuf,
             agg_sh):
    c = lax.axis_index("c")
    s = lax.axis_index("s")
    pltpu.sync_copy(src3.at[s], src_v)
    pltpu.sync_copy(dst3.at[s], dst_v)
    for jj in range(cp):
      j = c * cp + jj
      pltpu.sync_copy(zrows, agg_sh.at[pl.ds(s * ROWS_PER_TILE,
                                             ROWS_PER_TILE)])
      base = j * N

      def gix(t, _):
        i = t // (CHUNK // 16)
        k = t % (CHUNK // 16)
        gidx_v[i, pl.ds(k * 16, 16)] = src_v[i, pl.ds(k * 16, 16)] + base
        return 0

      lax.fori_loop(0, EPT_CH * (CHUNK // 16), gix, 0)
      plsc.subcore_barrier()

      def chunk_body(ch, _):
        pltpu.sync_copy(table.at[gidx_v.at[ch]], gbuf)
        pltpu.sync_copy(gbuf, agg_sh.at[dst_v.at[ch]], add=True)
        return 0

      lax.fori_loop(0, EPT_CH, chunk_body, 0)
      plsc.subcore_barrier()
      pltpu.sync_copy(
          agg_sh.at[pl.ds(s * ROWS_PER_TILE, ROWS_PER_TILE)],
          out_r.at[j, pl.ds(s * ROWS_PER_TILE, ROWS_PER_TILE)])

  return segsum


_segsum2 = _make_segsum(2)
_segsum8 = _make_segsum(8)


def _tc1_body(x_ref, agg_ref, w_ref, b_ref, out_ref):
  a = jnp.concatenate([agg_ref[0], agg_ref[1]], axis=-1)
  xa = x_ref[...] + a
  acc = jnp.dot(xa, w_ref[...], preferred_element_type=jnp.float32)
  out_ref[0] = jnp.maximum(acc + b_ref[0], 0.0)


def _tc1(x, agg1, w1, b1r):
  bn = 400
  grid = (N // bn, NHID // CW)
  return pl.pallas_call(
      _tc1_body,
      grid=grid,
      in_specs=[
          pl.BlockSpec((bn, NFEAT), lambda i, j: (i, 0)),
          pl.BlockSpec((2, bn, CW), lambda i, j: (0, i, 0)),
          pl.BlockSpec((NFEAT, CW), lambda i, j: (0, j)),
          pl.BlockSpec((1, 1, CW), lambda i, j: (j, 0, 0)),
      ],
      out_specs=pl.BlockSpec((1, bn, CW), lambda i, j: (j, i, 0)),
      out_shape=jax.ShapeDtypeStruct((NHID // CW, N, CW), jnp.float32),
  )(x, agg1, w1, b1r)


def _tc2_body(h_ref, agg_ref, w_ref, b_ref, out_ref):
  k = pl.program_id(1)

  @pl.when(k == 0)
  def _():
    out_ref[...] = jnp.broadcast_to(b_ref[...], out_ref.shape)

  ha = h_ref[0] + agg_ref[0]
  out_ref[...] += jnp.dot(ha, w_ref[...], preferred_element_type=jnp.float32)


def _tc2(h_r, agg2, w2, b2r):
  bn = 400
  grid = (N // bn, NHID // CW)
  return pl.pallas_call(
      _tc2_body,
      grid=grid,
      in_specs=[
          pl.BlockSpec((1, bn, CW), lambda i, k: (k, i, 0)),
          pl.BlockSpec((1, bn, CW), lambda i, k: (k, i, 0)),
          pl.BlockSpec((CW, NCLASS), lambda i, k: (k, 0)),
          pl.BlockSpec((1, NCLASS), lambda i, k: (0, 0)),
      ],
      out_specs=pl.BlockSpec((bn, NCLASS), lambda i, k: (i, 0)),
      out_shape=jax.ShapeDtypeStruct((N, NCLASS), jnp.float32),
  )(h_r, agg2, w2, b2r)


def kernel(x, edge_index, W1, b1, W2, b2):
  src = edge_index[0].astype(jnp.int32)
  dst = edge_index[1].astype(jnp.int32)
  pad = EPAD - E
  src3 = jnp.concatenate([src, jnp.zeros((pad,), jnp.int32)]).reshape(
      NTILES, EPT_CH, CHUNK)
  dst3 = jnp.concatenate([dst, jnp.full((pad,), N, jnp.int32)]).reshape(
      NTILES, EPT_CH, CHUNK)
  zrows = jnp.zeros((ROWS_PER_TILE, CW), jnp.float32)

  x2d = x.reshape(N, 2, CW).transpose(1, 0, 2).reshape(2 * N, CW)
  agg1 = _segsum2(x2d, src3, dst3, zrows)
  h_r = _tc1(x, agg1, W1, b1.reshape(NHID // CW, 1, CW))
  agg2 = _segsum8(h_r.reshape(8 * N, CW), src3, dst3, zrows)
  out = _tc2(h_r, agg2, W2, b2.reshape(1, NCLASS))
  return out


# speedup vs baseline: 3.8327x; 1.9036x over previous
import functools

import jax
import jax.numpy as jnp
from jax import lax
from jax.experimental import pallas as pl
from jax.experimental.pallas import tpu as pltpu
from jax.experimental.pallas import tpu_sc as plsc

N = 10000
E = 160000
NFEAT = 256
NHID = 1024
NCLASS = 256

NTILES = 16
NCORES = 2
CHUNK = 112
EPT_CH = 90
EPAD = NTILES * EPT_CH * CHUNK
NPAD = 10112
ROWS_PER_TILE = NPAD // NTILES
CW = 128


def _make_segsum(nchunks):
  cp = nchunks // NCORES
  nbuf = 3
  mesh = plsc.VectorSubcoreMesh(core_axis_name="c", subcore_axis_name="s")

  @functools.partial(
      pl.kernel,
      mesh=mesh,
      out_type=jax.ShapeDtypeStruct((nchunks, NPAD, CW), jnp.float32),
      scratch_types=[
          pltpu.VMEM((nbuf, 2, CHUNK), jnp.int32),
          pltpu.VMEM((nbuf, CHUNK), jnp.int32),
          pltpu.VMEM((nbuf, CHUNK), jnp.int32),
          [pltpu.VMEM((CHUNK, CW), jnp.float32) for _ in range(nbuf)],
          pltpu.VMEM_SHARED((NPAD, CW), jnp.float32),
          [pltpu.SemaphoreType.DMA for _ in range(nbuf)],
          [pltpu.SemaphoreType.DMA for _ in range(nbuf)],
          [pltpu.SemaphoreType.DMA for _ in range(nbuf)],
      ],
  )
  def segsum(table, e4, zrows, out_r, ebuf, gidx_v, dbuf, gbufs, agg_sh,
             se, sg, ss):
    c = lax.axis_index("c")
    s = lax.axis_index("s")

    def start_idx(ch, b):
      pltpu.async_copy(e4.at[s, ch], ebuf.at[b], se[b])

    def wait_idx(ch, b):
      pltpu.make_async_copy(e4.at[s, ch], ebuf.at[b], se[b]).wait()

    def start_gather(b):
      pltpu.async_copy(table.at[gidx_v.at[b]], gbufs[b], sg[b])

    def wait_gather(b):
      pltpu.make_async_copy(table.at[gidx_v.at[b]], gbufs[b], sg[b]).wait()

    def start_scatter(b):
      pltpu.async_copy(gbufs[b], agg_sh.at[dbuf.at[b]], ss[b], add=True)

    def wait_scatter(b):
      pltpu.make_async_copy(gbufs[b], agg_sh.at[dbuf.at[b]], ss[b]).wait()

    def unpack_idx(b, base):
      for k in range(CHUNK // 16):
        sl = pl.ds(k * 16, 16)
        gidx_v[b, sl] = ebuf[b, 0, sl] + base
        dbuf[b, sl] = ebuf[b, 1, sl]

    for jj in range(cp):
      j = c * cp + jj
      pltpu.sync_copy(zrows, agg_sh.at[pl.ds(s * ROWS_PER_TILE,
                                             ROWS_PER_TILE)])
      base = j * N
      plsc.subcore_barrier()

      start_idx(0, 0)
      start_idx(1, 1)
      wait_idx(0, 0)
      unpack_idx(0, base)
      start_gather(0)

      @pl.loop(0, EPT_CH, step=nbuf)
      def chunk_body(ch3):
        for bb in range(nbuf):
          ch = ch3 + bb
          b = bb
          b1 = (bb + 1) % nbuf
          b2 = (bb + 2) % nbuf

          @pl.when(ch + 1 < EPT_CH)
          def _():
            wait_idx(ch + 1, b1)

            @pl.when(ch >= 2)
            def _():
              wait_scatter(b1)

            unpack_idx(b1, base)
            start_gather(b1)

          @pl.when(ch + 2 < EPT_CH)
          def _():
            start_idx(ch + 2, b2)

          wait_gather(b)
          start_scatter(b)

      for ch in range(EPT_CH - 3, EPT_CH):
        wait_scatter(ch % nbuf)
      plsc.subcore_barrier()
      pltpu.sync_copy(
          agg_sh.at[pl.ds(s * ROWS_PER_TILE, ROWS_PER_TILE)],
          out_r.at[j, pl.ds(s * ROWS_PER_TILE, ROWS_PER_TILE)])

  return segsum


_segsum2 = _make_segsum(2)
_segsum8 = _make_segsum(8)


def _tc1_body(x_ref, agg_ref, w_ref, b_ref, out_ref):
  a = jnp.concatenate([agg_ref[0], agg_ref[1]], axis=-1)
  xa = x_ref[...] + a
  acc = jnp.dot(xa, w_ref[...], preferred_element_type=jnp.float32)
  out_ref[0] = jnp.maximum(acc + b_ref[0], 0.0)


def _tc1(x, agg1, w1, b1r):
  bn = 400
  grid = (N // bn, NHID // CW)
  return pl.pallas_call(
      _tc1_body,
      grid=grid,
      in_specs=[
          pl.BlockSpec((bn, NFEAT), lambda i, j: (i, 0)),
          pl.BlockSpec((2, bn, CW), lambda i, j: (0, i, 0)),
          pl.BlockSpec((NFEAT, CW), lambda i, j: (0, j)),
          pl.BlockSpec((1, 1, CW), lambda i, j: (j, 0, 0)),
      ],
      out_specs=pl.BlockSpec((1, bn, CW), lambda i, j: (j, i, 0)),
      out_shape=jax.ShapeDtypeStruct((NHID // CW, N, CW), jnp.float32),
  )(x, agg1, w1, b1r)


def _tc2_body(h_ref, agg_ref, w_ref, b_ref, out_ref):
  k = pl.program_id(1)

  @pl.when(k == 0)
  def _():
    out_ref[...] = jnp.broadcast_to(b_ref[...], out_ref.shape)

  ha = h_ref[0] + agg_ref[0]
  out_ref[...] += jnp.dot(ha, w_ref[...], preferred_element_type=jnp.float32)


def _tc2(h_r, agg2, w2, b2r):
  bn = 400
  grid = (N // bn, NHID // CW)
  return pl.pallas_call(
      _tc2_body,
      grid=grid,
      in_specs=[
          pl.BlockSpec((1, bn, CW), lambda i, k: (k, i, 0)),
          pl.BlockSpec((1, bn, CW), lambda i, k: (k, i, 0)),
          pl.BlockSpec((CW, NCLASS), lambda i, k: (k, 0)),
          pl.BlockSpec((1, NCLASS), lambda i, k: (0, 0)),
      ],
      out_specs=pl.BlockSpec((bn, NCLASS), lambda i, k: (i, 0)),
      out_shape=jax.ShapeDtypeStruct((N, NCLASS), jnp.float32),
  )(h_r, agg2, w2, b2r)


def kernel(x, edge_index, W1, b1, W2, b2):
  src = edge_index[0].astype(jnp.int32)
  dst = edge_index[1].astype(jnp.int32)
  pad = EPAD - E
  src3 = jnp.concatenate([src, jnp.zeros((pad,), jnp.int32)]).reshape(
      NTILES, EPT_CH, CHUNK)
  dst3 = jnp.concatenate([dst, jnp.full((pad,), N, jnp.int32)]).reshape(
      NTILES, EPT_CH, CHUNK)
  e4 = jnp.stack([src3, dst3], axis=2)
  zrows = jnp.zeros((ROWS_PER_TILE, CW), jnp.float32)

  x2d = x.reshape(N, 2, CW).transpose(1, 0, 2).reshape(2 * N, CW)
  agg1 = _segsum2(x2d, e4, zrows)
  h_r = _tc1(x, agg1, W1, b1.reshape(NHID // CW, 1, CW))
  agg2 = _segsum8(h_r.reshape(8 * N, CW), e4, zrows)
  out = _tc2(h_r, agg2, W2, b2.reshape(1, NCLASS))
  return out
